# Initial kernel scaffold; baseline (speedup 1.0000x reference)
import jax
import jax.numpy as jnp
from jax.experimental import pallas as pl


def kernel(node_feat, edge_index, dist, W_edge, W_node):
    raise NotImplementedError("write your pallas kernel here")


# speedup vs baseline: 2.6703x; 2.6703x over previous
import functools

import jax
import jax.numpy as jnp
from jax import lax
from jax.experimental import pallas as pl
from jax.experimental.pallas import tpu as pltpu
from jax.experimental.pallas import tpu_sc as plsc

_LEAK = 0.01
_DELTA = 0.5
_NS = 16
_NC = 2
_B = 80


def _stage1_body(nf_ref, w_ref, out_ref):
    out_ref[0] = jnp.dot(nf_ref[...], w_ref[...],
                         preferred_element_type=jnp.float32)


def _stage3_body(nf_ref, a0_ref, a1_ref, wa_ref, wb0_ref, wb1_ref, out_ref):
    acc = jnp.dot(nf_ref[...], wa_ref[...], preferred_element_type=jnp.float32)
    acc = acc + jnp.dot(a0_ref[...], wb0_ref[...],
                        preferred_element_type=jnp.float32)
    acc = acc + jnp.dot(a1_ref[...], wb1_ref[...],
                        preferred_element_type=jnp.float32)
    out_ref[...] = jnp.where(acc >= 0, acc, _LEAK * acc)


@functools.lru_cache(maxsize=None)
def _make_sc_edge(N, E, H):
    ept = E // _NS
    nch = ept // _B
    rpt = (N // _NS) & ~7
    tail = N - rpt * _NS
    nsl = H // 16
    mesh = plsc.VectorSubcoreMesh(core_axis_name="c", subcore_axis_name="s")

    @functools.partial(
        pl.kernel,
        out_type=jax.ShapeDtypeStruct((_NC, N, H), jnp.float32),
        mesh=mesh,
        scratch_types=[
            pltpu.VMEM((5, _B), jnp.int32),
            pltpu.VMEM((_B,), jnp.float32),
            pltpu.VMEM((H,), jnp.float32),
            pltpu.VMEM((_B, H), jnp.float32),
            pltpu.VMEM((_B, H), jnp.float32),
            pltpu.VMEM((_B, H), jnp.float32),
            pltpu.VMEM_SHARED((N, H), jnp.float32),
            pltpu.SemaphoreType.DMA,
            pltpu.SemaphoreType.DMA,
        ],
    )
    def sc_edge(T, meta, dd, b2, zrows, out,
                meta_v, dist_v, b_v, pa_v, pc_v, msg_v,
                acc_sh, sem0, sem1):
        c = lax.axis_index("c")
        s = lax.axis_index("s")
        pltpu.sync_copy(zrows.at[pl.ds(0, rpt)], acc_sh.at[pl.ds(s * rpt, rpt)])
        @pl.when(s == _NS - 1)
        def _zero_tail():
            pltpu.sync_copy(zrows.at[pl.ds(0, tail)],
                            acc_sh.at[pl.ds(rpt * _NS, tail)])
        pltpu.sync_copy(b2.at[c], b_v)
        plsc.subcore_barrier()
        b_vecs = [b_v[pl.ds(16 * k, 16)] for k in range(nsl)]

        def chunk_body(j, carry):
            pltpu.sync_copy(meta.at[s, j], meta_v)
            pltpu.sync_copy(dd.at[s, j], dist_v)
            cpa = pltpu.async_copy(T.at[meta_v.at[2 * c]], pa_v, sem0)
            cpb = pltpu.async_copy(T.at[meta_v.at[2 * c + 1]], pc_v, sem1)
            cpa.wait()
            cpb.wait()

            def group_body(g, carry2):
                dvec = dist_v[pl.ds(g * 16, 16)]
                cvec = jnp.where(dvec < _DELTA, jnp.float32(1.0),
                                 jnp.float32(0.0))
                for lane in range(16):
                    e = g * 16 + lane
                    d = dvec[lane]
                    coef = cvec[lane]
                    for k in range(nsl):
                        v = (pa_v[e, pl.ds(16 * k, 16)]
                             + pc_v[e, pl.ds(16 * k, 16)]
                             + d * b_vecs[k])
                        msg_v[e, pl.ds(16 * k, 16)] = (
                            jnp.where(v >= 0, v, _LEAK * v) * coef)
                return carry2

            lax.fori_loop(0, _B // 16, group_body, 0)
            pltpu.sync_copy(msg_v, acc_sh.at[meta_v.at[4]], add=True)
            return carry

        lax.fori_loop(0, nch, chunk_body, 0)
        plsc.subcore_barrier()
        pltpu.sync_copy(acc_sh.at[pl.ds(s * rpt, rpt)],
                        out.at[c, pl.ds(s * rpt, rpt)])
        @pl.when(s == _NS - 1)
        def _write_tail():
            pltpu.sync_copy(acc_sh.at[pl.ds(rpt * _NS, tail)],
                            out.at[c, pl.ds(rpt * _NS, tail)])

    return sc_edge


def kernel(node_feat, edge_index, dist, W_edge, W_node):
    N, IN = node_feat.shape
    OUT = W_node.shape[1]
    E = dist.shape[0]
    H = OUT // 2
    src = edge_index[0].astype(jnp.int32)
    dst = edge_index[1].astype(jnp.int32)
    A = W_edge[:IN]
    b = W_edge[IN]
    C = W_edge[IN + 1:]

    W2 = jnp.concatenate([A[:, :H], C[:, :H], A[:, H:], C[:, H:]], axis=1)
    m_blk = 1000
    P4 = pl.pallas_call(
        _stage1_body,
        grid=(N // m_blk, 4),
        in_specs=[
            pl.BlockSpec((m_blk, IN), lambda i, t: (i, 0)),
            pl.BlockSpec((IN, H), lambda i, t: (0, t)),
        ],
        out_specs=pl.BlockSpec((1, m_blk, H), lambda i, t: (t, i, 0)),
        out_shape=jax.ShapeDtypeStruct((4, N, H), jnp.float32),
    )(node_feat, W2)
    T = P4.reshape(4 * N, H)

    ept = E // _NS
    nch = ept // _B
    src3 = src.reshape(_NS, nch, _B)
    dst3 = dst.reshape(_NS, nch, _B)
    dd3 = dist.reshape(_NS, nch, _B)
    meta = jnp.stack(
        [src3, dst3 + N, src3 + 2 * N, dst3 + 3 * N, dst3], axis=2)
    b2 = b.reshape(_NC, H)
    zrows = jnp.zeros((N // _NS, H), jnp.float32)
    aggr2 = _make_sc_edge(N, E, H)(T, meta, dd3, b2, zrows)

    out = pl.pallas_call(
        _stage3_body,
        grid=(N // m_blk,),
        in_specs=[
            pl.BlockSpec((m_blk, IN), lambda i: (i, 0)),
            pl.BlockSpec((m_blk, H), lambda i: (i, 0)),
            pl.BlockSpec((m_blk, H), lambda i: (i, 0)),
            pl.BlockSpec((IN, OUT), lambda i: (0, 0)),
            pl.BlockSpec((H, OUT), lambda i: (0, 0)),
            pl.BlockSpec((H, OUT), lambda i: (0, 0)),
        ],
        out_specs=pl.BlockSpec((m_blk, OUT), lambda i: (i, 0)),
        out_shape=jax.ShapeDtypeStruct((N, OUT), jnp.float32),
    )(node_feat, aggr2[0], aggr2[1], W_node[:IN], W_node[IN:IN + H],
      W_node[IN + H:])
    return out
